# Initial kernel scaffold; baseline (speedup 1.0000x reference)
import jax
import jax.numpy as jnp
from jax.experimental import pallas as pl


def kernel(x):
    raise NotImplementedError("write your pallas kernel here")


# speedup vs baseline: 8.9901x; 8.9901x over previous
import math

import jax
import jax.numpy as jnp
from jax.experimental import pallas as pl
from jax.experimental.pallas import tpu as pltpu


def _body(x_ref, o_ref, *, k, c):
    x = x_ref[0]
    xb = jax.lax.bitcast_convert_type(x, jnp.int32)
    key = xb ^ (jnp.int32(0x7FFFFFFF) & (xb >> 31))

    p = x.shape[1]
    cnt0 = jnp.sum((key >= 0).astype(jnp.int32), axis=0, keepdims=True)
    prefix0 = jnp.where(cnt0 >= k, jnp.int32(0), jnp.int32(-2147483648))

    def step(i, prefix):
        bit = jnp.left_shift(jnp.int32(1), 30 - i)
        cand = prefix + bit
        cnt = jnp.sum((key >= cand).astype(jnp.int32), axis=0, keepdims=True)
        return jnp.where(cnt >= k, cand, prefix)

    kth = jax.lax.fori_loop(0, 31, step, prefix0)
    thr_bits = kth ^ (jnp.int32(0x7FFFFFFF) & (kth >> 31))
    thr = jax.lax.bitcast_convert_type(thr_bits, jnp.float32)

    out = jnp.where(x >= thr, x, jnp.float32(0.0))
    o_ref[0] = jnp.maximum(out, jnp.float32(0.0))


def kernel(x):
    b, c, h, w = x.shape
    n = h * w
    k = math.ceil(0.5 * c)
    xf = x.reshape(b, c, n)

    p = min(n, 512)
    grid = (b, n // p)
    import functools
    out = pl.pallas_call(
        functools.partial(_body, k=k, c=c),
        grid=grid,
        in_specs=[pl.BlockSpec((1, c, p), lambda i, j: (i, 0, j))],
        out_specs=pl.BlockSpec((1, c, p), lambda i, j: (i, 0, j)),
        out_shape=jax.ShapeDtypeStruct((b, c, n), jnp.float32),
    )(xf)
    return out.reshape(b, c, h, w)
